# baseline (speedup 1.0000x reference)
import functools

import jax
import jax.numpy as jnp
from jax import lax
from jax.experimental import pallas as pl
from jax.experimental.pallas import tpu as pltpu
from jax.experimental.pallas import tpu_sc as plsc

NUM_WORKERS = 32
SHORT = 32
DIM = 128


WORDS = SHORT // 2


def _sc_gather(ids_flat, table_i32, n_rows):
    b_per_w = n_rows // NUM_WORKERS
    mesh = plsc.VectorSubcoreMesh(core_axis_name="c", subcore_axis_name="s")

    @functools.partial(
        pl.kernel,
        mesh=mesh,
        out_type=jax.ShapeDtypeStruct((n_rows, WORDS), jnp.int32),
        scratch_types=[
            pltpu.VMEM((b_per_w,), jnp.int32),
            pltpu.VMEM((b_per_w, WORDS), jnp.int32),
            pltpu.SemaphoreType.DMA,
        ],
        compiler_params=pltpu.CompilerParams(use_tc_tiling_on_sc=False),
    )
    def gather_kernel(ids_hbm, table_hbm, out_hbm, idx_v, rows_v, sem):
        wid = lax.axis_index("s") * 2 + lax.axis_index("c")
        base = wid * b_per_w
        pltpu.sync_copy(ids_hbm.at[pl.ds(base, b_per_w)], idx_v)
        pltpu.async_copy(table_hbm.at[idx_v], rows_v, sem).wait()
        pltpu.sync_copy(rows_v, out_hbm.at[pl.ds(base, b_per_w)])

    return gather_kernel(ids_flat, table_i32)


def _proj_body(x_ref, w_ref, b_ref, o_ref):
    acc = jnp.dot(x_ref[...], w_ref[...], preferred_element_type=jnp.float32)
    o_ref[...] = (acc + b_ref[...]).astype(jnp.bfloat16)


def _tc_project(x, wt, b2, n_rows):
    block = 8192
    return pl.pallas_call(
        _proj_body,
        grid=(n_rows // block,),
        in_specs=[
            pl.BlockSpec((block, SHORT), lambda i: (i, 0)),
            pl.BlockSpec((SHORT, DIM), lambda i: (0, 0)),
            pl.BlockSpec((1, DIM), lambda i: (0, 0)),
        ],
        out_specs=pl.BlockSpec((block, DIM), lambda i: (i, 0)),
        out_shape=jax.ShapeDtypeStruct((n_rows, DIM), jnp.bfloat16),
    )(x, wt, b2)


def kernel(ids, embed, W, b):
    B, L = ids.shape
    n_rows = B * L
    num_emb = embed.shape[0]
    ids_flat = ids.reshape(n_rows).astype(jnp.int32)
    table_i32 = jax.lax.bitcast_convert_type(
        embed.reshape(num_emb, WORDS, 2), jnp.int32
    )
    x_i32 = _sc_gather(ids_flat, table_i32, n_rows)
    x = jax.lax.bitcast_convert_type(x_i32, jnp.bfloat16).reshape(n_rows, SHORT)
    wt = W.astype(jnp.bfloat16).T
    b2 = b.astype(jnp.bfloat16).reshape(1, DIM)
    out = _tc_project(x, wt, b2, n_rows)
    return out.reshape(B, L, DIM)
